# baseline (device time: 31253 ns/iter reference)
import jax
import jax.numpy as jnp
from jax import lax
from jax.experimental import pallas as pl
from jax.experimental.pallas import tpu as pltpu

N_DEV = 4
B, SQ, SKV, HQ, DH = 2, 128, 512, 16, 64
D_MODEL = 512
H_LOC = HQ // N_DEV
HD_LOC = H_LOC * DH
SKV_LOC = SKV // N_DEV

SRC_SLICES = {s: (0, SKV // N_DEV, s * (SKV // N_DEV)) for s in range(N_DEV)}
NKV = SKV

_DeviceIdType = getattr(pl, "DeviceIdType", None) or pltpu.DeviceIdType
MESH = _DeviceIdType.MESH
_sem_signal = getattr(pl, "semaphore_signal", None) or pltpu.semaphore_signal
_sem_wait = getattr(pl, "semaphore_wait", None) or pltpu.semaphore_wait
_CompilerParams = getattr(pltpu, "CompilerParams", None) or pltpu.TPUCompilerParams


def kernel(x, Wq, K_ext, V_ext, Wo):
    kT = jnp.transpose(K_ext, (0, 2, 3, 1)).reshape(B, HQ * DH, SKV_LOC)
    vT = jnp.transpose(V_ext, (0, 2, 3, 1)).reshape(B, HQ * DH, SKV_LOC)

    def body(x_ref, wq_ref, k_ref, v_ref, wo_ref, out_ref,
             kb_ref, vb_ref, ex_ref,
             local_sems, send_sems, krecv_sems, vrecv_sems,
             ex_send_sems, ex_recv_sems):
        my = lax.axis_index("i")

        barrier = pltpu.get_barrier_semaphore()
        for o in range(1, N_DEV):
            _sem_signal(barrier, inc=1, device_id=(lax.rem(my + o, N_DEV),),
                        device_id_type=MESH)
        _sem_wait(barrier, N_DEV - 1)

        def kv_rdma(src_dev, peer, ref, buf, sem, rsems):
            lo, ln, dst = SRC_SLICES[src_dev]
            return pltpu.make_async_remote_copy(
                src_ref=ref.at[:, peer * HD_LOC:(peer + 1) * HD_LOC, lo:lo + ln],
                dst_ref=buf.at[:, :, dst:dst + ln],
                send_sem=sem, recv_sem=rsems.at[src_dev],
                device_id=(peer,), device_id_type=MESH)

        def recv_wait(r, buf, rsems):
            rlo, rln, rdst = SRC_SLICES[r]
            pltpu.make_async_remote_copy(
                src_ref=buf.at[:, :, rdst:rdst + rln],
                dst_ref=buf.at[:, :, rdst:rdst + rln],
                send_sem=send_sems.at[0], recv_sem=rsems.at[r],
                device_id=(r,), device_id_type=MESH).wait_recv()

        for s in range(N_DEV):
            @pl.when(my == s)
            def _(s=s):
                lo, ln, dst = SRC_SLICES[s]
                peers = [p for p in range(N_DEV) if p != s]
                for i, p in enumerate(peers):
                    kv_rdma(s, p, k_ref, kb_ref, send_sems.at[2 * i],
                            krecv_sems).start()
                pltpu.make_async_copy(
                    k_ref.at[:, s * HD_LOC:(s + 1) * HD_LOC, lo:lo + ln],
                    kb_ref.at[:, :, dst:dst + ln], local_sems.at[0]).start()
                for i, p in enumerate(peers):
                    kv_rdma(s, p, v_ref, vb_ref, send_sems.at[2 * i + 1],
                            vrecv_sems).start()
                pltpu.make_async_copy(
                    v_ref.at[:, s * HD_LOC:(s + 1) * HD_LOC, lo:lo + ln],
                    vb_ref.at[:, :, dst:dst + ln], local_sems.at[1]).start()

        wqs = wq_ref[...] * jnp.float32(0.125)
        q = [jnp.dot(x_ref[b], wqs, preferred_element_type=jnp.float32)
             for b in range(B)]

        for s in range(N_DEV):
            @pl.when(my == s)
            def _(s=s):
                lo, ln, dst = SRC_SLICES[s]
                pltpu.make_async_copy(
                    k_ref.at[:, s * HD_LOC:(s + 1) * HD_LOC, lo:lo + ln],
                    kb_ref.at[:, :, dst:dst + ln], local_sems.at[0]).wait()
                for r in range(N_DEV):
                    if r != s:
                        recv_wait(r, kb_ref, krecv_sems)

        cb = lax.broadcasted_iota(jnp.int32, (NKV, SQ), 0) // 64
        qb = lax.broadcasted_iota(jnp.int32, (NKV, SQ), 1) // 64
        maskT = (qb == cb) | (cb == 0) | (lax.rem(qb + cb, 3) == 0)

        w_all = []
        for b in range(B):
            w_b = []
            for h in range(H_LOC):
                qh = q[b][:, h * DH:(h + 1) * DH]
                sT = lax.dot_general(
                    kb_ref[b][h * DH:(h + 1) * DH, :], qh,
                    (((0,), (1,)), ((), ())),
                    preferred_element_type=jnp.float32)
                ewT = jnp.where(maskT, jnp.exp(sT), jnp.float32(0.0))
                w_b.append(ewT / jnp.sum(ewT, axis=0, keepdims=True))
            w_all.append(w_b)

        for s in range(N_DEV):
            @pl.when(my == s)
            def _(s=s):
                lo, ln, dst = SRC_SLICES[s]
                pltpu.make_async_copy(
                    v_ref.at[:, s * HD_LOC:(s + 1) * HD_LOC, lo:lo + ln],
                    vb_ref.at[:, :, dst:dst + ln], local_sems.at[1]).wait()
                for r in range(N_DEV):
                    if r != s:
                        recv_wait(r, vb_ref, vrecv_sems)

        partners = [my + 1 - 2 * lax.rem(my, 2), 3 - my]
        H2 = D_MODEL // 2

        def exch(r, b, half):
            sl = slice(half * H2, (half + 1) * H2)
            idx = 4 * r + 2 * b + half
            return pltpu.make_async_remote_copy(
                src_ref=out_ref.at[b, :, sl], dst_ref=ex_ref.at[r, b, :, sl],
                send_sem=ex_send_sems.at[idx], recv_sem=ex_recv_sems.at[idx],
                device_id=(partners[(r + half) % 2],),
                device_id_type=MESH)

        e = {}
        for b in range(B):
            ctxT_b = jnp.concatenate(
                [lax.dot_general(
                    vb_ref[b][h * DH:(h + 1) * DH, :], w_all[b][h],
                    (((1,), (0,)), ((), ())),
                    preferred_element_type=jnp.float32)
                 for h in range(H_LOC)], axis=0)
            out_ref[b] = lax.dot_general(
                ctxT_b, wo_ref[...], (((0,), (0,)), ((), ())),
                preferred_element_type=jnp.float32)
            for half in range(2):
                e[(0, b, half)] = exch(0, b, half)
                e[(0, b, half)].start()
        for b in range(B):
            e[(0, b, 0)].wait()
            e[(0, b, 1)].wait()
            out_ref[b] = out_ref[b] + ex_ref[0, b]
            for half in range(2):
                e[(1, b, half)] = exch(1, b, half)
                e[(1, b, half)].start()
        for b in range(B):
            e[(1, b, 0)].wait()
            e[(1, b, 1)].wait()
            out_ref[b] = out_ref[b] + ex_ref[1, b]

        for s in range(N_DEV):
            @pl.when(my == s)
            def _(s=s):
                for i, p in enumerate([p for p in range(N_DEV) if p != s]):
                    kv_rdma(s, p, k_ref, kb_ref, send_sems.at[2 * i],
                            krecv_sems).wait_send()
                    kv_rdma(s, p, v_ref, vb_ref, send_sems.at[2 * i + 1],
                            vrecv_sems).wait_send()

    return pl.pallas_call(
        body,
        out_shape=jax.ShapeDtypeStruct((B, SQ, D_MODEL), jnp.float32),
        in_specs=[pl.BlockSpec(memory_space=pltpu.VMEM)] * 5,
        out_specs=pl.BlockSpec(memory_space=pltpu.VMEM),
        scratch_shapes=[
            pltpu.VMEM((B, HD_LOC, NKV), jnp.float32),
            pltpu.VMEM((B, HD_LOC, NKV), jnp.float32),
            pltpu.VMEM((2, B, SQ, D_MODEL), jnp.float32),
            pltpu.SemaphoreType.DMA((2,)),
            pltpu.SemaphoreType.DMA((6,)),
            pltpu.SemaphoreType.DMA((N_DEV,)),
            pltpu.SemaphoreType.DMA((N_DEV,)),
            pltpu.SemaphoreType.DMA((8,)),
            pltpu.SemaphoreType.DMA((8,)),
        ],
        compiler_params=_CompilerParams(collective_id=0),
    )(x, Wq, kT, vT, Wo)


# device time: 20085 ns/iter; 1.5560x vs baseline; 1.5560x over previous
import jax
import jax.numpy as jnp
from jax import lax
from jax.experimental import pallas as pl
from jax.experimental.pallas import tpu as pltpu

N_DEV = 4
B, SQ, SKV, HQ, DH = 2, 128, 512, 16, 64
D_MODEL = 512
H_LOC = HQ // N_DEV
HD_LOC = H_LOC * DH
SKV_LOC = SKV // N_DEV

SRC_SLICES = {0: (0, 128, 0), 1: (0, 128, 128), 2: (64, 64, 256), 3: (0, 64, 320)}
NKV = 384

_DeviceIdType = getattr(pl, "DeviceIdType", None) or pltpu.DeviceIdType
MESH = _DeviceIdType.MESH
_sem_signal = getattr(pl, "semaphore_signal", None) or pltpu.semaphore_signal
_sem_wait = getattr(pl, "semaphore_wait", None) or pltpu.semaphore_wait
_CompilerParams = getattr(pltpu, "CompilerParams", None) or pltpu.TPUCompilerParams


def kernel(x, Wq, K_ext, V_ext, Wo):
    kT = jnp.transpose(K_ext, (0, 2, 3, 1)).reshape(B, HQ * DH, SKV_LOC)
    v2 = V_ext.reshape(B, SKV_LOC, HQ * DH)

    def body(x_ref, wq_ref, k_ref, v_ref, wo_ref, out_ref,
             ks_ref, vs_ref, kb_ref, vb_ref, sb_ref, ex_ref, acc_ref,
             local_sems, send_sems, krecv_sems, vrecv_sems,
             ex_send_sems, ex_recv_sems):
        my = lax.axis_index("i")

        barrier = pltpu.get_barrier_semaphore()
        for o in range(1, N_DEV):
            _sem_signal(barrier, inc=1, device_id=(lax.rem(my + o, N_DEV),),
                        device_id_type=MESH)

        ks_ref[...] = k_ref[...].astype(jnp.bfloat16)
        vs_ref[...] = v_ref[...].astype(jnp.bfloat16)

        _sem_wait(barrier, N_DEV - 1)

        def k_rdma(src_dev, peer, sem):
            return pltpu.make_async_remote_copy(
                src_ref=ks_ref.at[:, peer * HD_LOC:(peer + 1) * HD_LOC, :],
                dst_ref=kb_ref.at[:, :, src_dev * SKV_LOC:(src_dev + 1) * SKV_LOC],
                send_sem=sem, recv_sem=krecv_sems.at[src_dev],
                device_id=(peer,), device_id_type=MESH)

        def k_recv_wait(r):
            sl = kb_ref.at[:, :, r * SKV_LOC:(r + 1) * SKV_LOC]
            pltpu.make_async_remote_copy(
                src_ref=sl, dst_ref=sl,
                send_sem=send_sems.at[0], recv_sem=krecv_sems.at[r],
                device_id=(r,), device_id_type=MESH).wait_recv()

        def v_rdma(src_dev, peer, sem):
            lo, ln, dst = SRC_SLICES[src_dev]
            return pltpu.make_async_remote_copy(
                src_ref=vs_ref.at[:, lo:lo + ln, peer * HD_LOC:(peer + 1) * HD_LOC],
                dst_ref=vb_ref.at[:, dst:dst + ln, :],
                send_sem=sem, recv_sem=vrecv_sems.at[src_dev],
                device_id=(peer,), device_id_type=MESH)

        def v_recv_wait(r):
            rlo, rln, rdst = SRC_SLICES[r]
            pltpu.make_async_remote_copy(
                src_ref=vb_ref.at[:, rdst:rdst + rln, :],
                dst_ref=vb_ref.at[:, rdst:rdst + rln, :],
                send_sem=send_sems.at[0], recv_sem=vrecv_sems.at[r],
                device_id=(r,), device_id_type=MESH).wait_recv()

        for s in range(N_DEV):
            @pl.when(my == s)
            def _(s=s):
                lo, ln, dst = SRC_SLICES[s]
                peers = [p for p in range(N_DEV) if p != s]
                for i, p in enumerate(peers):
                    k_rdma(s, p, send_sems.at[2 * i]).start()
                pltpu.make_async_copy(
                    ks_ref.at[:, s * HD_LOC:(s + 1) * HD_LOC, :],
                    kb_ref.at[:, :, s * SKV_LOC:(s + 1) * SKV_LOC],
                    local_sems.at[0]).start()
                for i, p in enumerate(peers):
                    v_rdma(s, p, send_sems.at[2 * i + 1]).start()
                pltpu.make_async_copy(
                    vs_ref.at[:, lo:lo + ln, s * HD_LOC:(s + 1) * HD_LOC],
                    vb_ref.at[:, dst:dst + ln, :], local_sems.at[1]).start()

        wqs = wq_ref[...] * jnp.float32(0.125)
        q = [jnp.dot(x_ref[b], wqs, preferred_element_type=jnp.float32)
             for b in range(B)]

        for s in range(N_DEV):
            @pl.when(my == s)
            def _(s=s):
                pltpu.make_async_copy(
                    ks_ref.at[:, s * HD_LOC:(s + 1) * HD_LOC, :],
                    kb_ref.at[:, :, s * SKV_LOC:(s + 1) * SKV_LOC],
                    local_sems.at[0]).wait()
                for r in range(N_DEV):
                    if r != s:
                        k_recv_wait(r)

        rb = lax.broadcasted_iota(jnp.int32, (SQ, NKV), 0) // 64
        c6 = lax.broadcasted_iota(jnp.int32, (SQ, NKV), 1) // 64
        cb = c6 + (c6 >= 4).astype(jnp.int32)
        mask = (rb == cb) | (cb == 0) | (lax.rem(rb + cb, 3) == 0)

        w_all = []
        for b in range(B):
            qb16 = q[b].astype(jnp.bfloat16)
            w_b = []
            for h in range(H_LOC):
                qh = qb16[:, h * DH:(h + 1) * DH]
                full = lax.dot_general(
                    qh, kb_ref[b][h * DH:(h + 1) * DH, :],
                    (((1,), (0,)), ((), ())),
                    preferred_element_type=jnp.float32)
                scores = jnp.concatenate(
                    [full[:, 0:256], full[:, 320:448]], axis=1)
                ew = jnp.where(mask, jnp.exp(scores), jnp.float32(0.0))
                w = ew / jnp.sum(ew, axis=1, keepdims=True)
                w_b.append(w.astype(jnp.bfloat16))
            w_all.append(w_b)

        for s in range(N_DEV):
            @pl.when(my == s)
            def _(s=s):
                lo, ln, dst = SRC_SLICES[s]
                pltpu.make_async_copy(
                    vs_ref.at[:, lo:lo + ln, s * HD_LOC:(s + 1) * HD_LOC],
                    vb_ref.at[:, dst:dst + ln, :], local_sems.at[1]).wait()
                for r in range(N_DEV):
                    if r != s:
                        v_recv_wait(r)

        partners = [my + 1 - 2 * lax.rem(my, 2), 3 - my]
        H2 = D_MODEL // 2

        def exch(r, b, half):
            sl = slice(half * H2, (half + 1) * H2)
            idx = 4 * r + 2 * b + half
            return pltpu.make_async_remote_copy(
                src_ref=sb_ref.at[b, :, sl], dst_ref=ex_ref.at[r, b, :, sl],
                send_sem=ex_send_sems.at[idx], recv_sem=ex_recv_sems.at[idx],
                device_id=(partners[(r + half) % 2],),
                device_id_type=MESH)

        e = {}
        for b in range(B):
            ctx_b = jnp.concatenate(
                [jnp.dot(w_all[b][h], vb_ref[b][:, h * DH:(h + 1) * DH],
                         preferred_element_type=jnp.float32)
                 for h in range(H_LOC)], axis=1)
            partial_b = jnp.dot(ctx_b, wo_ref[...],
                                preferred_element_type=jnp.float32)
            acc_ref[b] = partial_b
            sb_ref[b] = partial_b.astype(jnp.bfloat16)
            for half in range(2):
                e[(0, b, half)] = exch(0, b, half)
                e[(0, b, half)].start()
        for b in range(B):
            e[(0, b, 0)].wait()
            e[(0, b, 1)].wait()
            acc_b = acc_ref[b] + ex_ref[0, b].astype(jnp.float32)
            acc_ref[b] = acc_b
            sb_ref[b] = acc_b.astype(jnp.bfloat16)
            for half in range(2):
                e[(1, b, half)] = exch(1, b, half)
                e[(1, b, half)].start()
        out_dmas = []
        for b in range(B):
            e[(1, b, 0)].wait()
            e[(1, b, 1)].wait()
            acc_ref[b] = acc_ref[b] + ex_ref[1, b].astype(jnp.float32)
            d = pltpu.make_async_copy(acc_ref.at[b], out_ref.at[b],
                                      local_sems.at[b])
            d.start()
            out_dmas.append(d)
        for d in out_dmas:
            d.wait()

        for s in range(N_DEV):
            @pl.when(my == s)
            def _(s=s):
                for i, p in enumerate([p for p in range(N_DEV) if p != s]):
                    k_rdma(s, p, send_sems.at[2 * i]).wait_send()
                    v_rdma(s, p, send_sems.at[2 * i + 1]).wait_send()

    return pl.pallas_call(
        body,
        out_shape=jax.ShapeDtypeStruct((B, SQ, D_MODEL), jnp.float32),
        in_specs=[pl.BlockSpec(memory_space=pltpu.VMEM)] * 5,
        out_specs=pl.BlockSpec(memory_space=pltpu.MemorySpace.HBM),
        scratch_shapes=[
            pltpu.VMEM((B, HQ * DH, SKV_LOC), jnp.bfloat16),
            pltpu.VMEM((B, SKV_LOC, HQ * DH), jnp.bfloat16),
            pltpu.VMEM((B, HD_LOC, SKV), jnp.bfloat16),
            pltpu.VMEM((B, NKV, HD_LOC), jnp.bfloat16),
            pltpu.VMEM((B, SQ, D_MODEL), jnp.bfloat16),
            pltpu.VMEM((2, B, SQ, D_MODEL), jnp.bfloat16),
            pltpu.VMEM((B, SQ, D_MODEL), jnp.float32),
            pltpu.SemaphoreType.DMA((2,)),
            pltpu.SemaphoreType.DMA((6,)),
            pltpu.SemaphoreType.DMA((N_DEV,)),
            pltpu.SemaphoreType.DMA((N_DEV,)),
            pltpu.SemaphoreType.DMA((8,)),
            pltpu.SemaphoreType.DMA((8,)),
        ],
        compiler_params=_CompilerParams(collective_id=0),
    )(x, Wq, kT, v2, Wo)
